# Initial kernel scaffold; baseline (speedup 1.0000x reference)
import jax
import jax.numpy as jnp
from jax.experimental import pallas as pl


def kernel(z, edge_index, node_to_subgraph, node_to_original_node, subgraph_to_graph, z_emb, tbc_W, tbc_b, conv_W, gru_Wih, gru_Whh, gru_bih, gru_bhh, pool_W1, pool_b1, pool_W2, pool_b2, sg_W, sg_b, cg_W, cg_b, fc1_W, fc1_b, fc2_W, fc2_b, fc3_W, fc3_b):
    raise NotImplementedError("write your pallas kernel here")


# speedup vs baseline: 4.2379x; 4.2379x over previous
import functools

import jax
import jax.numpy as jnp
from jax import lax
from jax.experimental import pallas as pl
from jax.experimental.pallas import tpu as pltpu
from jax.experimental.pallas import tpu_sc as plsc

N = 50000
S = 5000
G = 64
E = 800000
D = 64
NL = 5

NC = 2
NS = 16
CH = 120

BN = 2000
NBLK = N // BN
BS = 1000
SBLK = S // BS

EPT = E // NS
ETP = 50400
ECHP = ETP // CH
EGB = 20
EGRP = ECHP // EGB
DUMP = N

NTCH = 26
NTB = NTCH * CH
NT_MAIN = NS * NTB
NTAIL = N - NT_MAIN

SCH_M = 41
ST_OFF = SCH_M * CH
STAIL = S - ST_OFF

ZR = 3128
ACC_ROWS = NS * ZR
SACC = 5040
BF = 500
FS = S // BF

_F32 = jnp.float32


def _dot(a, b):
    return jnp.dot(a, b, preferred_element_type=_F32)


def _zf_body(z_ref, emb_ref, out_ref):
    zb = z_ref[0, 0]
    oh = (zb[:, None] == lax.broadcasted_iota(jnp.int32, (1, 100), 1)
          ).astype(_F32)
    for t in range(NL + 1):
        out_ref[t] = _dot(oh, emb_ref[t])


def _zf_all(z3, z_emb):
    return pl.pallas_call(
        _zf_body,
        grid=(NBLK,),
        in_specs=[
            pl.BlockSpec((1, 1, BN), lambda i: (i, 0, 0)),
            pl.BlockSpec((NL + 1, 100, D), lambda i: (0, 0, 0)),
        ],
        out_specs=pl.BlockSpec((NL + 1, BN, D), lambda i: (0, i, 0)),
        out_shape=jax.ShapeDtypeStruct((NL + 1, N, D), _F32),
    )(z3, z_emb)


def _first_body(n2s_ref, out_ref, lt_ref, eq_ref):
    si = pl.program_id(0)
    ni = pl.program_id(1)

    @pl.when(ni == 0)
    def _():
        lt_ref[...] = jnp.zeros_like(lt_ref)
        eq_ref[...] = jnp.zeros_like(eq_ref)

    nb = n2s_ref[0, 0]
    sids = si * BF + lax.broadcasted_iota(jnp.int32, (BF, 1), 0)
    lt_ref[...] += jnp.sum((nb[None, :] < sids).astype(jnp.int32), axis=1,
                           keepdims=True)
    eq_ref[...] += jnp.sum((nb[None, :] == sids).astype(jnp.int32), axis=1,
                           keepdims=True)

    @pl.when(ni == NBLK - 1)
    def _():
        fr = jnp.where(eq_ref[...] == 0, N, lt_ref[...])
        out_ref[0, 0] = jnp.minimum(fr, N - 1)[:, 0]


def _first_idx(n2s3):
    out = pl.pallas_call(
        _first_body,
        grid=(FS, NBLK),
        in_specs=[pl.BlockSpec((1, 1, BN), lambda i, j: (j, 0, 0))],
        out_specs=pl.BlockSpec((1, 1, BF), lambda i, j: (i, 0, 0)),
        out_shape=jax.ShapeDtypeStruct((FS, 1, BF), jnp.int32),
        scratch_shapes=[pltpu.VMEM((BF, 1), jnp.int32),
                        pltpu.VMEM((BF, 1), jnp.int32)],
    )(n2s3)
    return out.reshape(S)


def _pre_body(xin_ref, zf_ref, tbcW_ref, tbcb_ref, convW_ref, whhT_ref,
              bhh_ref, xr_ref, m2_ref, gh_ref):
    zfb = zf_ref[...]
    if xin_ref is None:
        xb = zfb
    else:
        xb = (_dot(xin_ref[0], tbcW_ref[0:32])
              + _dot(xin_ref[1], tbcW_ref[32:64])
              + _dot(zfb, tbcW_ref[64:128]) + tbcb_ref[...])
    m = _dot(xb, convW_ref[...])
    xr_ref[...] = xb
    m2_ref[0] = m[:, 0:32]
    m2_ref[1] = m[:, 32:64]
    gh_ref[...] = _dot(xb, whhT_ref[...]) + bhh_ref[...]


_PRE_OUT = [
    jax.ShapeDtypeStruct((N, D), _F32),
    jax.ShapeDtypeStruct((NC, N, 32), _F32),
    jax.ShapeDtypeStruct((N, 3 * D), _F32),
]
_PRE_OUT_SPECS = [
    pl.BlockSpec((BN, D), lambda i: (i, 0)),
    pl.BlockSpec((NC, BN, 32), lambda i: (0, i, 0)),
    pl.BlockSpec((BN, 3 * D), lambda i: (i, 0)),
]


def _pre0(zf, convW, whhT, bhh):
    def b(zf_ref, convW_ref, whhT_ref, bhh_ref, xr_ref, m2_ref, gh_ref):
        _pre_body(None, zf_ref, None, None, convW_ref, whhT_ref, bhh_ref,
                  xr_ref, m2_ref, gh_ref)

    return pl.pallas_call(
        b,
        grid=(NBLK,),
        in_specs=[
            pl.BlockSpec((BN, D), lambda i: (i, 0)),
            pl.BlockSpec((D, D), lambda i: (0, 0)),
            pl.BlockSpec((D, 3 * D), lambda i: (0, 0)),
            pl.BlockSpec((1, 3 * D), lambda i: (0, 0)),
        ],
        out_specs=_PRE_OUT_SPECS,
        out_shape=_PRE_OUT,
    )(zf, convW, whhT, bhh)


def _pre(xin2, zf, tbcW, tbcb, convW, whhT, bhh):
    return pl.pallas_call(
        _pre_body,
        grid=(NBLK,),
        in_specs=[
            pl.BlockSpec((NC, BN, 32), lambda i: (0, i, 0)),
            pl.BlockSpec((BN, D), lambda i: (i, 0)),
            pl.BlockSpec((2 * D, D), lambda i: (0, 0)),
            pl.BlockSpec((1, D), lambda i: (0, 0)),
            pl.BlockSpec((D, D), lambda i: (0, 0)),
            pl.BlockSpec((D, 3 * D), lambda i: (0, 0)),
            pl.BlockSpec((1, 3 * D), lambda i: (0, 0)),
        ],
        out_specs=_PRE_OUT_SPECS,
        out_shape=_PRE_OUT,
    )(xin2, zf, tbcW, tbcb, convW, whhT, bhh)


def _post_body(agg2_ref, xr_ref, gh_ref, zf_ref, wihT_ref, bih_ref, sgW_ref,
               sgb_ref, cgW_ref, cgb_ref, ysub_ref, yctx_ref, xn_ref):
    agg = jnp.concatenate([agg2_ref[0], agg2_ref[1]], axis=-1)
    gi = _dot(agg, wihT_ref[...]) + bih_ref[...]
    gh = gh_ref[...]
    x = xr_ref[...]
    r = jax.nn.sigmoid(gi[:, 0:D] + gh[:, 0:D])
    zg = jax.nn.sigmoid(gi[:, D:2 * D] + gh[:, D:2 * D])
    n = jnp.tanh(gi[:, 2 * D:3 * D] + r * gh[:, 2 * D:3 * D])
    xn = (1.0 - zg) * n + zg * x
    zfb = zf_ref[...]
    gs = jax.nn.sigmoid(_dot(zfb, sgW_ref[...]) + sgb_ref[...])
    gc = jax.nn.sigmoid(_dot(zfb, cgW_ref[...]) + cgb_ref[...])
    ys = gs * xn
    yc = gc * xn
    ysub_ref[0] = ys[:, 0:32]
    ysub_ref[1] = ys[:, 32:64]
    yctx_ref[0] = yc[:, 0:32]
    yctx_ref[1] = yc[:, 32:64]
    xn_ref[0] = xn[:, 0:32]
    xn_ref[1] = xn[:, 32:64]


def _post(agg2, xr, gh, zf, wihT, bih, sgW, sgb, cgW, cgb):
    split_spec = pl.BlockSpec((NC, BN, 32), lambda i: (0, i, 0))
    return pl.pallas_call(
        _post_body,
        grid=(NBLK,),
        in_specs=[
            split_spec,
            pl.BlockSpec((BN, D), lambda i: (i, 0)),
            pl.BlockSpec((BN, 3 * D), lambda i: (i, 0)),
            pl.BlockSpec((BN, D), lambda i: (i, 0)),
            pl.BlockSpec((D, 3 * D), lambda i: (0, 0)),
            pl.BlockSpec((1, 3 * D), lambda i: (0, 0)),
            pl.BlockSpec((D, D), lambda i: (0, 0)),
            pl.BlockSpec((1, D), lambda i: (0, 0)),
            pl.BlockSpec((D, D), lambda i: (0, 0)),
            pl.BlockSpec((1, D), lambda i: (0, 0)),
        ],
        out_specs=[split_spec, split_spec, split_spec],
        out_shape=[jax.ShapeDtypeStruct((NC, N, 32), _F32)] * 3,
    )(agg2, xr, gh, zf, wihT, bih, sgW, sgb, cgW, cgb)


def _fgate_body(x2_ref, zf_ref, sgW_ref, sgb_ref, cgW_ref, cgb_ref,
                ysub_ref, yctx_ref):
    x = jnp.concatenate([x2_ref[0], x2_ref[1]], axis=-1)
    zfb = zf_ref[...]
    gs = jax.nn.sigmoid(_dot(zfb, sgW_ref[...]) + sgb_ref[...])
    gc = jax.nn.sigmoid(_dot(zfb, cgW_ref[...]) + cgb_ref[...])
    ys = gs * x
    yc = gc * x
    ysub_ref[0] = ys[:, 0:32]
    ysub_ref[1] = ys[:, 32:64]
    yctx_ref[0] = yc[:, 0:32]
    yctx_ref[1] = yc[:, 32:64]


def _fgates(x2, zf, sgW, sgb, cgW, cgb):
    split_spec = pl.BlockSpec((NC, BN, 32), lambda i: (0, i, 0))
    return pl.pallas_call(
        _fgate_body,
        grid=(NBLK,),
        in_specs=[
            split_spec,
            pl.BlockSpec((BN, D), lambda i: (i, 0)),
            pl.BlockSpec((D, D), lambda i: (0, 0)),
            pl.BlockSpec((1, D), lambda i: (0, 0)),
            pl.BlockSpec((D, D), lambda i: (0, 0)),
            pl.BlockSpec((1, D), lambda i: (0, 0)),
        ],
        out_specs=[split_spec, split_spec],
        out_shape=[jax.ShapeDtypeStruct((NC, N, 32), _F32)] * 2,
    )(x2, zf, sgW, sgb, cgW, cgb)


def _poolmlp_body(xsub_ref, xcen_ref, xctx_ref, W1_ref, b1_ref, W2_ref,
                  b2_ref, h2_ref):
    xs = jnp.concatenate([xsub_ref[0], xsub_ref[1]], axis=-1)
    xc = jnp.concatenate([xcen_ref[0], xcen_ref[1]], axis=-1)
    xx = jnp.concatenate([xctx_ref[0], xctx_ref[1]], axis=-1)
    pre = (_dot(xs, W1_ref[0:D]) + _dot(xc, W1_ref[D:2 * D])
           + _dot(xx, W1_ref[2 * D:3 * D]) + b1_ref[...])
    h = _dot(jax.nn.relu(pre), W2_ref[...]) + b2_ref[...]
    h2_ref[0] = h[:, 0:32]
    h2_ref[1] = h[:, 32:64]


def _poolmlp(xsub2, xcen2, xctx2, W1, b1, W2, b2):
    split_spec = pl.BlockSpec((NC, BS, 32), lambda i: (0, i, 0))
    return pl.pallas_call(
        _poolmlp_body,
        grid=(SBLK,),
        in_specs=[
            split_spec, split_spec, split_spec,
            pl.BlockSpec((3 * D, D), lambda i: (0, 0)),
            pl.BlockSpec((1, D), lambda i: (0, 0)),
            pl.BlockSpec((D, D), lambda i: (0, 0)),
            pl.BlockSpec((1, D), lambda i: (0, 0)),
        ],
        out_specs=split_spec,
        out_shape=jax.ShapeDtypeStruct((NC, S, 32), _F32),
    )(xsub2, xcen2, xctx2, W1, b1, W2, b2)


def _final_body(h2_ref, s2g_ref, fc1W_ref, fc1b_ref, fc2W_ref, fc2b_ref,
                fc3W_ref, fc3b_ref, out_ref, acc_ref):
    i = pl.program_id(0)

    @pl.when(i == 0)
    def _():
        acc_ref[...] = jnp.zeros_like(acc_ref)

    h = jnp.concatenate([h2_ref[0], h2_ref[1]], axis=-1)
    sg = s2g_ref[0, 0]
    oh = (sg[:, None] == lax.broadcasted_iota(jnp.int32, (1, G), 1)
          ).astype(_F32)
    acc_ref[...] += lax.dot_general(oh, h, (((0,), (0,)), ((), ())),
                                    preferred_element_type=_F32)

    @pl.when(i == SBLK - 1)
    def _():
        def elu(v):
            return jnp.where(v > 0, v, jnp.exp(jnp.minimum(v, 0.0)) - 1.0)

        y = acc_ref[...]
        y = elu(_dot(y, fc1W_ref[...]) + fc1b_ref[...])
        y = elu(_dot(y, fc2W_ref[...]) + fc2b_ref[...])
        out_ref[...] = _dot(y, fc3W_ref[...]) + fc3b_ref[...]


def _final(h2, s2g3, fc1W, fc1b, fc2W, fc2b, fc3W, fc3b):
    return pl.pallas_call(
        _final_body,
        grid=(SBLK,),
        in_specs=[
            pl.BlockSpec((NC, BS, 32), lambda i: (0, i, 0)),
            pl.BlockSpec((1, 1, BS), lambda i: (i, 0, 0)),
            pl.BlockSpec((D, 32), lambda i: (0, 0)),
            pl.BlockSpec((1, 32), lambda i: (0, 0)),
            pl.BlockSpec((32, 16), lambda i: (0, 0)),
            pl.BlockSpec((1, 16), lambda i: (0, 0)),
            pl.BlockSpec((16, 1), lambda i: (0, 0)),
            pl.BlockSpec((1, 1), lambda i: (0, 0)),
        ],
        out_specs=pl.BlockSpec((G, 1), lambda i: (0, 0)),
        out_shape=jax.ShapeDtypeStruct((G, 1), _F32),
        scratch_shapes=[pltpu.VMEM((G, D), _F32)],
    )(h2, s2g3, fc1W, fc1b, fc2W, fc2b, fc3W, fc3b)


def _edge_agg_body(m2, src2, dst2, zrows, out, src_v, dst_v, rows_a, rows_b,
                   acc, sem_a, sem_b):
    c = lax.axis_index("c")
    s = lax.axis_index("s")
    pltpu.sync_copy(zrows, acc.at[pl.ds(s * ZR, ZR)])
    plsc.subcore_barrier()

    @pl.loop(0, EGRP)
    def _(g):
        pltpu.sync_copy(src2.at[s].at[pl.ds(g * EGB, EGB)], src_v)
        pltpu.sync_copy(dst2.at[s].at[pl.ds(g * EGB, EGB)], dst_v)

        @pl.loop(0, EGB, step=2)
        def _(j):
            ga = pltpu.async_copy(m2.at[c].at[src_v.at[j]], rows_a, sem_a)
            gb = pltpu.async_copy(m2.at[c].at[src_v.at[j + 1]], rows_b,
                                  sem_b)
            ga.wait()
            pltpu.sync_copy(rows_a, acc.at[dst_v.at[j]], add=True)
            gb.wait()
            pltpu.sync_copy(rows_b, acc.at[dst_v.at[j + 1]], add=True)

    plsc.subcore_barrier()

    @pl.when(s < NS - 1)
    def _():
        pltpu.sync_copy(acc.at[pl.ds(s * ZR, ZR)],
                        out.at[c].at[pl.ds(s * ZR, ZR)])

    @pl.when(s == NS - 1)
    def _():
        last = N - (NS - 1) * ZR
        pltpu.sync_copy(acc.at[pl.ds((NS - 1) * ZR, last)],
                        out.at[c].at[pl.ds((NS - 1) * ZR, last)])


def _pool_body(ysub2, yctx2, xn2, n2s2, n2on2, n2st, n2ont, first2, firstt,
               zrows, outsub, outcen, outctx, idx_sub, idx_ctx, tidx, rows,
               cen_idx, cen_rows, acc_sub, acc_ctx, sem):
    c = lax.axis_index("c")
    s = lax.axis_index("s")

    @pl.when(s < 10)
    def _():
        pltpu.sync_copy(zrows.at[pl.ds(0, 504)],
                        acc_sub.at[pl.ds(s * 504, 504)])
        pltpu.sync_copy(zrows.at[pl.ds(0, 504)],
                        acc_ctx.at[pl.ds(s * 504, 504)])

    pltpu.sync_copy(n2s2.at[s], idx_sub)
    pltpu.sync_copy(n2on2.at[s], idx_ctx)

    @pl.when(s == NS - 1)
    def _():
        pltpu.sync_copy(n2st, tidx.at[pl.ds(0, 1)])
        pltpu.sync_copy(n2ont, tidx.at[pl.ds(1, 1)])

    plsc.subcore_barrier()
    base = s * NTB

    @pl.loop(0, NTCH)
    def _(j):
        pltpu.sync_copy(ysub2.at[c].at[pl.ds(base + j * CH, CH)], rows)
        pltpu.sync_copy(rows, acc_sub.at[idx_sub.at[j]], add=True)
        pltpu.sync_copy(yctx2.at[c].at[pl.ds(base + j * CH, CH)], rows)
        pltpu.sync_copy(rows, acc_ctx.at[idx_ctx.at[j]], add=True)

    @pl.when(s == NS - 1)
    def _():
        tr = rows.at[pl.ds(0, NTAIL)]
        pltpu.sync_copy(ysub2.at[c].at[pl.ds(NT_MAIN, NTAIL)], tr)
        pltpu.sync_copy(tr, acc_sub.at[tidx.at[0]], add=True)
        pltpu.sync_copy(yctx2.at[c].at[pl.ds(NT_MAIN, NTAIL)], tr)
        pltpu.sync_copy(tr, acc_ctx.at[tidx.at[1]], add=True)

    @pl.loop(0, 3)
    def _(k):
        jj = s + NS * k

        @pl.when(jj < SCH_M)
        def _():
            pltpu.sync_copy(first2.at[jj], cen_idx)
            pltpu.async_copy(xn2.at[c].at[cen_idx], cen_rows, sem).wait()
            pltpu.sync_copy(cen_rows, outcen.at[c].at[pl.ds(jj * CH, CH)])

        @pl.when(jj == SCH_M)
        def _():
            ci = cen_idx.at[pl.ds(0, STAIL)]
            cr = cen_rows.at[pl.ds(0, STAIL)]
            pltpu.sync_copy(firstt.at[0], ci)
            pltpu.async_copy(xn2.at[c].at[ci], cr, sem).wait()
            pltpu.sync_copy(cr, outcen.at[c].at[pl.ds(ST_OFF, STAIL)])

    plsc.subcore_barrier()

    @pl.when(s < 9)
    def _():
        pltpu.sync_copy(acc_sub.at[pl.ds(s * 504, 504)],
                        outsub.at[c].at[pl.ds(s * 504, 504)])
        pltpu.sync_copy(acc_ctx.at[pl.ds(s * 504, 504)],
                        outctx.at[c].at[pl.ds(s * 504, 504)])

    @pl.when(s == 9)
    def _():
        last = S - 9 * 504
        pltpu.sync_copy(acc_sub.at[pl.ds(9 * 504, last)],
                        outsub.at[c].at[pl.ds(9 * 504, last)])
        pltpu.sync_copy(acc_ctx.at[pl.ds(9 * 504, last)],
                        outctx.at[c].at[pl.ds(9 * 504, last)])


def _gather_body(h2, n2on2, n2ont, out, idx_v, tidx, rows, sem):
    c = lax.axis_index("c")
    s = lax.axis_index("s")
    pltpu.sync_copy(n2on2.at[s], idx_v)
    base = s * NTB

    @pl.loop(0, NTCH)
    def _(j):
        pltpu.async_copy(h2.at[c].at[idx_v.at[j]], rows, sem).wait()
        pltpu.sync_copy(rows, out.at[c].at[pl.ds(base + j * CH, CH)])

    @pl.when(s == NS - 1)
    def _():
        tr = rows.at[pl.ds(0, NTAIL)]
        pltpu.sync_copy(n2ont, tidx)
        pltpu.async_copy(h2.at[c].at[tidx.at[0]], tr, sem).wait()
        pltpu.sync_copy(tr, out.at[c].at[pl.ds(NT_MAIN, NTAIL)])


@functools.lru_cache(maxsize=None)
def _sc_kernels():
    mesh = plsc.VectorSubcoreMesh(core_axis_name="c", subcore_axis_name="s",
                                  num_cores=NC, num_subcores=NS)
    params = pltpu.CompilerParams(use_tc_tiling_on_sc=False)
    edge = pl.kernel(
        _edge_agg_body,
        out_type=jax.ShapeDtypeStruct((NC, N, 32), _F32),
        mesh=mesh,
        compiler_params=params,
        scratch_types=[
            pltpu.VMEM((EGB, CH), jnp.int32),
            pltpu.VMEM((EGB, CH), jnp.int32),
            pltpu.VMEM((CH, 32), _F32),
            pltpu.VMEM((CH, 32), _F32),
            pltpu.VMEM_SHARED((ACC_ROWS, 32), _F32),
            pltpu.SemaphoreType.DMA,
            pltpu.SemaphoreType.DMA,
        ],
    )
    pool = pl.kernel(
        _pool_body,
        out_type=[jax.ShapeDtypeStruct((NC, S, 32), _F32)] * 3,
        mesh=mesh,
        compiler_params=params,
        scratch_types=[
            pltpu.VMEM((NTCH, CH), jnp.int32),
            pltpu.VMEM((NTCH, CH), jnp.int32),
            pltpu.VMEM((2, NTAIL), jnp.int32),
            pltpu.VMEM((CH, 32), _F32),
            pltpu.VMEM((CH,), jnp.int32),
            pltpu.VMEM((CH, 32), _F32),
            pltpu.VMEM_SHARED((SACC, 32), _F32),
            pltpu.VMEM_SHARED((SACC, 32), _F32),
            pltpu.SemaphoreType.DMA,
        ],
    )
    gather = pl.kernel(
        _gather_body,
        out_type=jax.ShapeDtypeStruct((NC, N, 32), _F32),
        mesh=mesh,
        compiler_params=params,
        scratch_types=[
            pltpu.VMEM((NTCH, CH), jnp.int32),
            pltpu.VMEM((1, NTAIL), jnp.int32),
            pltpu.VMEM((CH, 32), _F32),
            pltpu.SemaphoreType.DMA,
        ],
    )
    return edge, pool, gather


def _edge_agg(m2, src2, dst2, zrows):
    return _sc_kernels()[0](m2, src2, dst2, zrows)


def _pool(ysub2, yctx2, xn2, idxs, zrows):
    n2s2, n2st, n2on2, n2ont, first2, firstt = idxs
    return _sc_kernels()[1](ysub2, yctx2, xn2, n2s2, n2on2, n2st, n2ont,
                            first2, firstt, zrows)


def _gather(h2, idxs):
    return _sc_kernels()[2](h2, idxs[2], idxs[3])


def kernel(z, edge_index, node_to_subgraph, node_to_original_node,
           subgraph_to_graph, z_emb, tbc_W, tbc_b, conv_W, gru_Wih, gru_Whh,
           gru_bih, gru_bhh, pool_W1, pool_b1, pool_W2, pool_b2, sg_W, sg_b,
           cg_W, cg_b, fc1_W, fc1_b, fc2_W, fc2_b, fc3_W, fc3_b):
    z3 = z.astype(jnp.int32).reshape(NBLK, 1, BN)
    n2s = node_to_subgraph.astype(jnp.int32)
    n2on = node_to_original_node.astype(jnp.int32)
    pad_e = ETP - EPT
    src2 = jnp.pad(edge_index[0].astype(jnp.int32).reshape(NS, EPT),
                   ((0, 0), (0, pad_e))).reshape(NS, ECHP, CH)
    dst2 = jnp.pad(edge_index[1].astype(jnp.int32).reshape(NS, EPT),
                   ((0, 0), (0, pad_e)),
                   constant_values=DUMP).reshape(NS, ECHP, CH)
    n2s2 = n2s[:NT_MAIN].reshape(NS, NTCH, CH)
    n2st = n2s[NT_MAIN:].reshape(1, NTAIL)
    n2on2 = n2on[:NT_MAIN].reshape(NS, NTCH, CH)
    n2ont = n2on[NT_MAIN:].reshape(1, NTAIL)
    s2g3 = subgraph_to_graph.astype(jnp.int32).reshape(SBLK, 1, BS)
    zrows = jnp.zeros((ZR, 32), _F32)

    zf_all = _zf_all(z3, z_emb)
    first = _first_idx(n2s.reshape(NBLK, 1, BN))
    first2 = first[:ST_OFF].reshape(SCH_M, CH)
    firstt = first[ST_OFF:].reshape(1, STAIL)
    idxs = (n2s2, n2st, n2on2, n2ont, first2, firstt)
    whhT = jnp.swapaxes(gru_Whh, 1, 2)
    wihT = jnp.swapaxes(gru_Wih, 1, 2)

    x2 = None
    for l in range(NL):
        zf = zf_all[l]
        if l == 0:
            xr, m2, gh = _pre0(zf, conv_W[0], whhT[0],
                               gru_bhh[0].reshape(1, -1))
        else:
            xr, m2, gh = _pre(x2, zf, tbc_W[l], tbc_b[l].reshape(1, -1),
                              conv_W[l], whhT[l], gru_bhh[l].reshape(1, -1))
        agg2 = _edge_agg(m2, src2, dst2, zrows)
        ysub2, yctx2, xn2 = _post(agg2, xr, gh, zf, wihT[l],
                                  gru_bih[l].reshape(1, -1), sg_W[l],
                                  sg_b[l].reshape(1, -1), cg_W[l],
                                  cg_b[l].reshape(1, -1))
        xsub2, xcen2, xctx2 = _pool(ysub2, yctx2, xn2, idxs, zrows)
        h2 = _poolmlp(xsub2, xcen2, xctx2, pool_W1[l],
                      pool_b1[l].reshape(1, -1), pool_W2[l],
                      pool_b2[l].reshape(1, -1))
        x2 = _gather(h2, idxs)

    zf = zf_all[NL]
    ysub2, yctx2 = _fgates(x2, zf, sg_W[NL], sg_b[NL].reshape(1, -1),
                           cg_W[NL], cg_b[NL].reshape(1, -1))
    xsub2, xcen2, xctx2 = _pool(ysub2, yctx2, x2, idxs, zrows)
    h2 = _poolmlp(xsub2, xcen2, xctx2, pool_W1[NL], pool_b1[NL].reshape(1, -1),
                  pool_W2[NL], pool_b2[NL].reshape(1, -1))
    return _final(h2, s2g3, fc1_W, fc1_b.reshape(1, -1), fc2_W,
                  fc2_b.reshape(1, -1), fc3_W, fc3_b.reshape(1, -1))
